# baseline (speedup 1.0000x reference)
import functools

import jax
import jax.numpy as jnp
from jax import lax
from jax.experimental import pallas as pl
from jax.experimental.pallas import tpu as pltpu
from jax.experimental.pallas import tpu_sc as plsc

NUSER = 1000000
NITEM = 100000
HID = 64
BATCH = 4096

_info = plsc.get_sparse_core_info()
_NC, _NS, _NL = _info.num_cores, _info.num_subcores, _info.num_lanes
_NW = _NC * _NS
_BPW = BATCH // _NW
_NBUF = 4


@functools.partial(
    pl.kernel,
    mesh=plsc.VectorSubcoreMesh(core_axis_name="c", subcore_axis_name="s"),
    out_type=jax.ShapeDtypeStruct((HID, BATCH), jnp.float32),
    scratch_types=(
        [pltpu.VMEM((_BPW,), jnp.int32)]
        + [pltpu.VMEM((HID, 128), jnp.float32)] * _NBUF
        + [pltpu.VMEM((HID, _BPW), jnp.float32)]
        + [pltpu.SemaphoreType.DMA]
    ),
    compiler_params=pltpu.CompilerParams(
        needs_layout_passes=False, disable_bounds_checks=True
    ),
)
def _user_lookup(user_hbm, uft_hbm, uout_hbm,
                 uidx_v, blk0, blk1, blk2, blk3, cols_v, sem):
    blks = (blk0, blk1, blk2, blk3)
    wid = lax.axis_index("s") * _NC + lax.axis_index("c")
    base = pl.multiple_of(wid * _BPW, _BPW)
    pltpu.sync_copy(user_hbm.at[pl.ds(base, _BPW)], uidx_v)
    lanes = lax.iota(jnp.int32, _NL)

    def extract(blk, rr, r):
        rr_v = jnp.full((_NL,), rr, jnp.int32)
        r_v = jnp.full((_NL,), r, jnp.int32)
        for k in range(HID // _NL):
            cvec = k * _NL + lanes
            val = plsc.load_gather(blk, [cvec, rr_v])
            plsc.store_scatter(cols_v, [cvec, r_v], val)

    def group(g):
        vec = uidx_v[pl.ds(g * 16, 16)]
        for h in range(16 // _NBUF):
            handles = []
            for b in range(_NBUF):
                r0 = pl.multiple_of((vec[h * _NBUF + b] >> 7) * 128, 128)
                handles.append(pltpu.async_copy(
                    uft_hbm.at[:, pl.ds(r0, 128)], blks[b], sem))
            for b in range(_NBUF):
                j = h * _NBUF + b
                handles[b].wait()
                extract(blks[b], vec[j] & 127, g * 16 + j)

    pl.loop(0, _BPW // 16)(group)
    pltpu.async_copy(cols_v, uout_hbm.at[:, pl.ds(base, _BPW)], sem).wait()


@functools.partial(
    pl.kernel,
    mesh=plsc.VectorSubcoreMesh(core_axis_name="c", subcore_axis_name="s"),
    out_type=jax.ShapeDtypeStruct((BATCH, HID), jnp.float32),
    scratch_types=[
        pltpu.VMEM((_BPW,), jnp.int32),
        pltpu.VMEM((_BPW, HID), jnp.float32),
        pltpu.SemaphoreType.DMA,
    ],
    compiler_params=pltpu.CompilerParams(use_tc_tiling_on_sc=False),
)
def _item_lookup(item_hbm, if_hbm, iout_hbm, iidx_v, rows_v, sem):
    wid = lax.axis_index("s") * _NC + lax.axis_index("c")
    base = wid * _BPW
    pltpu.sync_copy(item_hbm.at[pl.ds(base, _BPW)], iidx_v)
    pltpu.async_copy(if_hbm.at[iidx_v], rows_v, sem).wait()
    pltpu.async_copy(rows_v, iout_hbm.at[pl.ds(base, _BPW)], sem).wait()


def kernel(user, item, user_factor, item_factor):
    user = user.astype(jnp.int32)
    item = item.astype(jnp.int32)
    uout_t = _user_lookup(user, user_factor.T)
    item_emb = _item_lookup(item, item_factor)
    return (uout_t.T, item_emb)
